# baseline (speedup 1.0000x reference)
import functools

import jax
import jax.numpy as jnp
from jax import lax
from jax.experimental import pallas as pl
from jax.experimental.pallas import tpu as pltpu
from jax.experimental.pallas import tpu_sc as plsc

_G = 16
_EBLK = 4096


def _proj_body(t_ref, w_ref, b_ref, out_ref):
    x = lax.dot_general(
        t_ref[...], w_ref[...],
        (((0,), (1,)), ((), ())),
        preferred_element_type=jnp.float32,
    )
    y = (x + b_ref[...]).astype(jnp.bfloat16)
    yi = lax.bitcast_convert_type(y, jnp.uint16)
    y3 = yi.reshape(yi.shape[0] // 2, 2, yi.shape[1])
    lo = y3[:, 0, :].astype(jnp.uint32)
    hi = y3[:, 1, :].astype(jnp.uint32)
    out_ref[...] = lax.bitcast_convert_type((hi << 16) | lo, jnp.int32)


def _proj_table(table_t, W, b2d):
    D, V = table_t.shape
    O = W.shape[0]
    return pl.pallas_call(
        _proj_body,
        grid=((V + _EBLK - 1) // _EBLK,),
        in_specs=[
            pl.BlockSpec((D, _EBLK), lambda i: (0, i)),
            pl.BlockSpec((O, D), lambda i: (0, 0)),
            pl.BlockSpec((1, O), lambda i: (0, 0)),
        ],
        out_specs=pl.BlockSpec((_EBLK // 2, O), lambda i: (i, 0)),
        out_shape=jax.ShapeDtypeStruct((V // 2, O), jnp.int32),
    )(table_t, W, b2d)


def _make_gather(B, D, num_cores, num_subcores):
    nw = num_cores * num_subcores
    b_per_w = B // nw
    n_groups = b_per_w // _G
    mesh = plsc.VectorSubcoreMesh(core_axis_name="c", subcore_axis_name="s")

    @functools.partial(
        pl.kernel,
        mesh=mesh,
        compiler_params=pltpu.CompilerParams(
            needs_layout_passes=False, use_tc_tiling_on_sc=True
        ),
        out_type=jax.ShapeDtypeStruct((B, D), jnp.int32),
        scratch_types=[
            pltpu.VMEM((b_per_w,), jnp.int32),
            pltpu.VMEM((b_per_w, D), jnp.int32),
            pltpu.SemaphoreType.DMA,
            pltpu.SemaphoreType.DMA,
        ],
    )
    def gather(idx_hbm, proj_hbm, out_hbm, idx_v, rows_v, isem, sem):
        wid = lax.axis_index("s") * num_cores + lax.axis_index("c")
        base = wid * b_per_w
        pltpu.async_copy(idx_hbm.at[pl.ds(base, b_per_w)], idx_v, isem).wait()

        def fire(g):
            v = idx_v[pl.ds(g * _G, _G)]
            return [
                pltpu.async_copy(
                    proj_hbm.at[v[i] >> 1],
                    rows_v.at[g * _G + i],
                    sem,
                )
                for i in range(_G)
            ]

        pending = fire(0)
        for g in range(n_groups):
            nxt = fire(g + 1) if g + 1 < n_groups else []
            for c in pending:
                c.wait()
            pending = nxt

        pltpu.async_copy(rows_v, out_hbm.at[pl.ds(base, b_per_w)], isem).wait()

    return gather


def kernel(indices, table, W, b):
    info = plsc.get_sparse_core_info()
    V, D = table.shape
    O = W.shape[0]
    B = indices.shape[0]
    proj = _proj_table(table.T, W, b.reshape(1, -1))
    idx32 = indices.astype(jnp.int32)
    packed = _make_gather(B, O, info.num_cores, info.num_subcores)(
        idx32, proj
    )
    out_bf = lax.bitcast_convert_type(packed, jnp.bfloat16)
    sel = jnp.where((idx32 & 1)[:, None] == 1, out_bf[..., 1], out_bf[..., 0])
    return sel.astype(jnp.float32)


# speedup vs baseline: 1.7925x; 1.7925x over previous
import functools

import jax
import jax.numpy as jnp
from jax import lax
from jax.experimental import pallas as pl
from jax.experimental.pallas import tpu as pltpu
from jax.experimental.pallas import tpu_sc as plsc

_G = 16
_EBLK = 4096


def _proj_body(t_ref, w_ref, b_ref, out_ref):
    x = lax.dot_general(
        t_ref[...], w_ref[...],
        (((0,), (1,)), ((), ())),
        preferred_element_type=jnp.float32,
    )
    out_ref[...] = x + b_ref[...]


def _proj_table(table_t, W, b2d):
    D, V = table_t.shape
    O = W.shape[0]
    return pl.pallas_call(
        _proj_body,
        grid=((V + _EBLK - 1) // _EBLK,),
        in_specs=[
            pl.BlockSpec((D, _EBLK), lambda i: (0, i)),
            pl.BlockSpec((O, D), lambda i: (0, 0)),
            pl.BlockSpec((1, O), lambda i: (0, 0)),
        ],
        out_specs=pl.BlockSpec((_EBLK, O), lambda i: (i, 0)),
        out_shape=jax.ShapeDtypeStruct((V, O), jnp.float32),
    )(table_t, W, b2d)


def _make_gather(B, D, num_cores, num_subcores):
    nw = num_cores * num_subcores
    b_per_w = B // nw
    n_groups = b_per_w // _G
    mesh = plsc.VectorSubcoreMesh(core_axis_name="c", subcore_axis_name="s")

    @functools.partial(
        pl.kernel,
        mesh=mesh,
        compiler_params=pltpu.CompilerParams(
            needs_layout_passes=False, use_tc_tiling_on_sc=True
        ),
        out_type=jax.ShapeDtypeStruct((B, D), jnp.float32),
        scratch_types=[
            pltpu.VMEM((b_per_w,), jnp.int32),
            pltpu.VMEM((b_per_w, D), jnp.float32),
            pltpu.SemaphoreType.DMA,
            pltpu.SemaphoreType.DMA,
        ],
    )
    def gather(idx_hbm, proj_hbm, out_hbm, idx_v, rows_v, isem, sem):
        wid = lax.axis_index("s") * num_cores + lax.axis_index("c")
        base = wid * b_per_w
        pltpu.async_copy(idx_hbm.at[pl.ds(base, b_per_w)], idx_v, isem).wait()

        def fire(g):
            v = idx_v[pl.ds(g * _G, _G)]
            return [
                pltpu.async_copy(
                    proj_hbm.at[v[i]],
                    rows_v.at[g * _G + i],
                    sem,
                )
                for i in range(_G)
            ]

        pending = fire(0)
        for g in range(n_groups):
            nxt = fire(g + 1) if g + 1 < n_groups else []
            for c in pending:
                c.wait()
            pending = nxt

        pltpu.async_copy(rows_v, out_hbm.at[pl.ds(base, b_per_w)], isem).wait()

    return gather


def kernel(indices, table, W, b):
    info = plsc.get_sparse_core_info()
    V, D = table.shape
    O = W.shape[0]
    B = indices.shape[0]
    proj = _proj_table(table.T, W, b.reshape(1, -1))
    return _make_gather(B, O, info.num_cores, info.num_subcores)(
        indices.astype(jnp.int32), proj
    )
